# Initial kernel scaffold; baseline (speedup 1.0000x reference)
import functools

import jax
import jax.numpy as jnp
import numpy as np
from jax import lax
from jax.experimental import pallas as pl
from jax.experimental.pallas import tpu as pltpu
from jax.experimental.pallas import tpu_sc as plsc

N = 10000
E = 320000
D = 128
H = 14
HP = 16
LATENT = 7
SOFTPLUS_INV_1 = float(np.log(np.expm1(1.0)))

NC = 2
NS = 16
NW = NC * NS
EDGES_PER_WORKER = E // NW
CHUNK = 80
NCHUNK = EDGES_PER_WORKER // CHUNK
ROWS_PER_TILE = N // NS

MROWS = 1000
GRID = N // MROWS


def _dense_body(x_ref, w_ref, b_ref, o_ref):
    o_ref[...] = (
        jnp.dot(x_ref[...], w_ref[...], preferred_element_type=jnp.float32)
        + b_ref[...]
    )


def _combine_body(p_ref, w_ref, b_ref, o_ref):
    a = jnp.maximum(p_ref[0] + p_ref[1], 0.0)
    o_ref[...] = (
        jnp.dot(a, w_ref[...], preferred_element_type=jnp.float32) + b_ref[...]
    )


def _head_body(p_ref, w_ref, b_ref, o_ref):
    a = jnp.maximum(p_ref[0] + p_ref[1], 0.0)
    lat = jnp.dot(a, w_ref[...], preferred_element_type=jnp.float32) + b_ref[...]
    z = lat + SOFTPLUS_INV_1
    sp = jnp.maximum(z, 0.0) + jnp.log1p(jnp.exp(-jnp.abs(z)))
    col = lax.broadcasted_iota(jnp.int32, lat.shape, 1)
    o_ref[...] = jnp.where(col < LATENT, lat, sp)


def _dense1(x, w, b):
    return pl.pallas_call(
        _dense_body,
        grid=(GRID,),
        in_specs=[
            pl.BlockSpec((MROWS, D), lambda i: (i, 0)),
            pl.BlockSpec((D, HP), lambda i: (0, 0)),
            pl.BlockSpec((1, HP), lambda i: (0, 0)),
        ],
        out_specs=pl.BlockSpec((MROWS, HP), lambda i: (i, 0)),
        out_shape=jax.ShapeDtypeStruct((N, HP), jnp.float32),
    )(x, w, b)


def _tc_stage(body, p, w, b):
    return pl.pallas_call(
        body,
        grid=(GRID,),
        in_specs=[
            pl.BlockSpec((NC, MROWS, HP), lambda i: (0, i, 0)),
            pl.BlockSpec((HP, HP), lambda i: (0, 0)),
            pl.BlockSpec((1, HP), lambda i: (0, 0)),
        ],
        out_specs=pl.BlockSpec((MROWS, HP), lambda i: (i, 0)),
        out_shape=jax.ShapeDtypeStruct((N, HP), jnp.float32),
    )(p, w, b)


def _spmm_body(h_hbm, src_hbm, dst_hbm, w_hbm, out_hbm,
               acc, src_v, dst_v, w_v, rows_v, sem):
    c = lax.axis_index("c")
    s = lax.axis_index("s")
    wid = c * NS + s

    pltpu.sync_copy(src_hbm.at[wid], src_v)
    pltpu.sync_copy(dst_hbm.at[wid], dst_v)
    pltpu.sync_copy(w_hbm.at[wid], w_v)

    def _zero(j, carry):
        rows_v[j, :] = jnp.zeros((HP,), jnp.float32)
        return carry

    lax.fori_loop(0, CHUNK, _zero, 0)
    full = ROWS_PER_TILE // CHUNK
    rem = ROWS_PER_TILE - full * CHUNK
    base_rows = s * ROWS_PER_TILE
    for r in range(full):
        pltpu.sync_copy(rows_v, acc.at[pl.ds(base_rows + r * CHUNK, CHUNK)])
    pltpu.sync_copy(rows_v.at[pl.ds(0, rem)],
                    acc.at[pl.ds(base_rows + full * CHUNK, rem)])
    plsc.subcore_barrier()

    def _chunk(i, carry):
        pltpu.async_copy(h_hbm.at[src_v.at[i]], rows_v, sem).wait()
        for j in range(CHUNK):
            rows_v[j, :] = rows_v[j, :] * w_v[i, j]
        pltpu.sync_copy(rows_v, acc.at[dst_v.at[i]], add=True)
        return carry

    lax.fori_loop(0, NCHUNK, _chunk, 0)
    plsc.subcore_barrier()

    pltpu.sync_copy(acc.at[pl.ds(base_rows, ROWS_PER_TILE)],
                    out_hbm.at[c, pl.ds(base_rows, ROWS_PER_TILE)])


@functools.partial(
    pl.kernel,
    mesh=plsc.VectorSubcoreMesh(core_axis_name="c", subcore_axis_name="s"),
    out_type=jax.ShapeDtypeStruct((NC, N, HP), jnp.float32),
    scratch_types=[
        pltpu.VMEM_SHARED((N, HP), jnp.float32),
        pltpu.VMEM((NCHUNK, CHUNK), jnp.int32),
        pltpu.VMEM((NCHUNK, CHUNK), jnp.int32),
        pltpu.VMEM((NCHUNK, CHUNK), jnp.float32),
        pltpu.VMEM((CHUNK, HP), jnp.float32),
        pltpu.SemaphoreType.DMA,
    ],
)
def _spmm(h_hbm, src_hbm, dst_hbm, w_hbm, out_hbm,
          acc, src_v, dst_v, w_v, rows_v, sem):
    _spmm_body(h_hbm, src_hbm, dst_hbm, w_hbm, out_hbm,
               acc, src_v, dst_v, w_v, rows_v, sem)


def kernel(x, edge_index, edge_weight, W1, b1, W2, b2, Wd, bd):
    f32 = jnp.float32
    W1p = jnp.zeros((D, HP), f32).at[:, :H].set(W1)
    b1p = jnp.zeros((1, HP), f32).at[0, :H].set(b1)
    W2p = jnp.zeros((HP, HP), f32).at[:H, :H].set(W2)
    b2p = jnp.zeros((1, HP), f32).at[0, :H].set(b2)
    Wdp = jnp.zeros((HP, HP), f32).at[:H, : 2 * LATENT].set(Wd)
    bdp = jnp.zeros((1, HP), f32).at[0, : 2 * LATENT].set(bd)

    src3 = edge_index[0].reshape(NW, NCHUNK, CHUNK)
    dst3 = edge_index[1].reshape(NW, NCHUNK, CHUNK)
    w3 = edge_weight.reshape(NW, NCHUNK, CHUNK)

    h1 = _dense1(x, W1p, b1p)
    p1 = _spmm(h1, src3, dst3, w3)
    h2 = _tc_stage(_combine_body, p1, W2p, b2p)
    p2 = _spmm(h2, src3, dst3, w3)
    out = _tc_stage(_head_body, p2, Wdp, bdp)
    return out[:, :LATENT], out[:, LATENT : 2 * LATENT]


# speedup vs baseline: 17.0448x; 17.0448x over previous
import functools

import jax
import jax.numpy as jnp
import numpy as np
from jax import lax
from jax.experimental import pallas as pl
from jax.experimental.pallas import tpu as pltpu
from jax.experimental.pallas import tpu_sc as plsc

N = 10000
E = 320000
D = 128
H = 14
HP = 16
LATENT = 7
SOFTPLUS_INV_1 = float(np.log(np.expm1(1.0)))

NC = 2
NS = 16
NW = NC * NS
EDGES_PER_WORKER = E // NW
CHUNK = 80
NCHUNK = EDGES_PER_WORKER // CHUNK
NP = 10240
ROWS_PER_TILE = NP // NS

MROWS = 1000
GRID = N // MROWS


def _dense_body(x_ref, w_ref, b_ref, o_ref):
    o_ref[...] = (
        jnp.dot(x_ref[...], w_ref[...], preferred_element_type=jnp.float32)
        + b_ref[...]
    )


def _combine_body(p_ref, w_ref, b_ref, o_ref):
    a = jnp.maximum(p_ref[0] + p_ref[1], 0.0)
    o_ref[...] = (
        jnp.dot(a, w_ref[...], preferred_element_type=jnp.float32) + b_ref[...]
    )


def _head_body(p_ref, w_ref, b_ref, o_ref):
    a = jnp.maximum(p_ref[0] + p_ref[1], 0.0)
    lat = jnp.dot(a, w_ref[...], preferred_element_type=jnp.float32) + b_ref[...]
    z = lat + SOFTPLUS_INV_1
    sp = jnp.maximum(z, 0.0) + jnp.log1p(jnp.exp(-jnp.abs(z)))
    col = lax.broadcasted_iota(jnp.int32, lat.shape, 1)
    o_ref[...] = jnp.where(col < LATENT, lat, sp)


def _dense1(x, w, b):
    return pl.pallas_call(
        _dense_body,
        grid=(GRID,),
        in_specs=[
            pl.BlockSpec((MROWS, D), lambda i: (i, 0)),
            pl.BlockSpec((D, HP), lambda i: (0, 0)),
            pl.BlockSpec((1, HP), lambda i: (0, 0)),
        ],
        out_specs=pl.BlockSpec((MROWS, HP), lambda i: (i, 0)),
        out_shape=jax.ShapeDtypeStruct((NP, HP), jnp.float32),
    )(x, w, b)


def _tc_stage(body, p, w, b):
    return pl.pallas_call(
        body,
        grid=(GRID,),
        in_specs=[
            pl.BlockSpec((NC, MROWS, HP), lambda i: (0, i, 0)),
            pl.BlockSpec((HP, HP), lambda i: (0, 0)),
            pl.BlockSpec((1, HP), lambda i: (0, 0)),
        ],
        out_specs=pl.BlockSpec((MROWS, HP), lambda i: (i, 0)),
        out_shape=jax.ShapeDtypeStruct((NP, HP), jnp.float32),
    )(p, w, b)


def _spmm_body(h_hbm, src_hbm, dst_hbm, w_hbm, out_hbm,
               h_sh, acc, src_v, dst_v, w_v, rows_v, sem):
    c = lax.axis_index("c")
    s = lax.axis_index("s")
    wid = c * NS + s
    base_rows = s * ROWS_PER_TILE

    pltpu.sync_copy(src_hbm.at[wid], src_v)
    pltpu.sync_copy(dst_hbm.at[wid], dst_v)
    pltpu.sync_copy(w_hbm.at[wid], w_v)

    pltpu.sync_copy(h_hbm.at[pl.ds(base_rows, ROWS_PER_TILE)],
                    h_sh.at[pl.ds(base_rows, ROWS_PER_TILE)])

    def _zero(j, carry):
        rows_v[j, :] = jnp.zeros((HP,), jnp.float32)
        return carry

    lax.fori_loop(0, CHUNK, _zero, 0)
    for r in range(ROWS_PER_TILE // CHUNK):
        pltpu.sync_copy(rows_v, acc.at[pl.ds(base_rows + r * CHUNK, CHUNK)])
    plsc.subcore_barrier()

    def _chunk(i, carry):
        pltpu.async_copy(h_sh.at[src_v.at[i]], rows_v, sem).wait()
        for j16 in range(0, CHUNK, HP):
            w16 = w_v[i, pl.ds(j16, HP)]
            for jj in range(HP):
                rows_v[j16 + jj, :] = rows_v[j16 + jj, :] * w16[jj]
        pltpu.sync_copy(rows_v, acc.at[dst_v.at[i]], add=True)
        return carry

    lax.fori_loop(0, NCHUNK, _chunk, 0)
    plsc.subcore_barrier()

    pltpu.sync_copy(acc.at[pl.ds(base_rows, ROWS_PER_TILE)],
                    out_hbm.at[c, pl.ds(base_rows, ROWS_PER_TILE)])


@functools.partial(
    pl.kernel,
    mesh=plsc.VectorSubcoreMesh(core_axis_name="c", subcore_axis_name="s"),
    compiler_params=pltpu.CompilerParams(use_tc_tiling_on_sc=False),
    out_type=jax.ShapeDtypeStruct((NC, NP, HP), jnp.float32),
    scratch_types=[
        pltpu.VMEM_SHARED((NP, HP), jnp.float32),
        pltpu.VMEM_SHARED((NP, HP), jnp.float32),
        pltpu.VMEM((NCHUNK, CHUNK), jnp.int32),
        pltpu.VMEM((NCHUNK, CHUNK), jnp.int32),
        pltpu.VMEM((NCHUNK, CHUNK), jnp.float32),
        pltpu.VMEM((CHUNK, HP), jnp.float32),
        pltpu.SemaphoreType.DMA,
    ],
)
def _spmm(h_hbm, src_hbm, dst_hbm, w_hbm, out_hbm,
          h_sh, acc, src_v, dst_v, w_v, rows_v, sem):
    _spmm_body(h_hbm, src_hbm, dst_hbm, w_hbm, out_hbm,
               h_sh, acc, src_v, dst_v, w_v, rows_v, sem)


def kernel(x, edge_index, edge_weight, W1, b1, W2, b2, Wd, bd):
    f32 = jnp.float32
    W1p = jnp.zeros((D, HP), f32).at[:, :H].set(W1)
    b1p = jnp.zeros((1, HP), f32).at[0, :H].set(b1)
    W2p = jnp.zeros((HP, HP), f32).at[:H, :H].set(W2)
    b2p = jnp.zeros((1, HP), f32).at[0, :H].set(b2)
    Wdp = jnp.zeros((HP, HP), f32).at[:H, : 2 * LATENT].set(Wd)
    bdp = jnp.zeros((1, HP), f32).at[0, : 2 * LATENT].set(bd)

    src3 = edge_index[0].reshape(NW, NCHUNK, CHUNK)
    dst3 = edge_index[1].reshape(NW, NCHUNK, CHUNK)
    w3 = edge_weight.reshape(NW, NCHUNK, CHUNK)

    h1 = _dense1(x, W1p, b1p)
    p1 = _spmm(h1, src3, dst3, w3)
    h2 = _tc_stage(_combine_body, p1, W2p, b2p)
    p2 = _spmm(h2, src3, dst3, w3)
    out = _tc_stage(_head_body, p2, Wdp, bdp)
    return out[:N, :LATENT], out[:N, LATENT : 2 * LATENT]


# speedup vs baseline: 19.3733x; 1.1366x over previous
import functools

import jax
import jax.numpy as jnp
import numpy as np
from jax import lax
from jax.experimental import pallas as pl
from jax.experimental.pallas import tpu as pltpu
from jax.experimental.pallas import tpu_sc as plsc

N = 10000
E = 320000
D = 128
H = 14
HP = 16
LATENT = 7
SOFTPLUS_INV_1 = float(np.log(np.expm1(1.0)))

NC = 2
NS = 16
NW = NC * NS
EDGES_PER_WORKER = E // NW
CHUNK = 128
NCHUNK = 80
EPW_PAD = NCHUNK * CHUNK
NP = 10240
ROWS_PER_TILE = NP // NS

MROWS = 1000
GRID = N // MROWS


def _dense_body(x_ref, w_ref, b_ref, o_ref):
    o_ref[...] = (
        jnp.dot(x_ref[...], w_ref[...], preferred_element_type=jnp.float32)
        + b_ref[...]
    )


def _combine_body(p_ref, w_ref, b_ref, o_ref):
    a = jnp.maximum(p_ref[0] + p_ref[1], 0.0)
    o_ref[...] = (
        jnp.dot(a, w_ref[...], preferred_element_type=jnp.float32) + b_ref[...]
    )


def _head_body(p_ref, w_ref, b_ref, o_ref):
    a = jnp.maximum(p_ref[0] + p_ref[1], 0.0)
    lat = jnp.dot(a, w_ref[...], preferred_element_type=jnp.float32) + b_ref[...]
    z = lat + SOFTPLUS_INV_1
    sp = jnp.maximum(z, 0.0) + jnp.log1p(jnp.exp(-jnp.abs(z)))
    col = lax.broadcasted_iota(jnp.int32, lat.shape, 1)
    o_ref[...] = jnp.where(col < LATENT, lat, sp)


def _dense1(x, w, b):
    return pl.pallas_call(
        _dense_body,
        grid=(GRID,),
        in_specs=[
            pl.BlockSpec((MROWS, D), lambda i: (i, 0)),
            pl.BlockSpec((D, HP), lambda i: (0, 0)),
            pl.BlockSpec((1, HP), lambda i: (0, 0)),
        ],
        out_specs=pl.BlockSpec((MROWS, HP), lambda i: (i, 0)),
        out_shape=jax.ShapeDtypeStruct((NP, HP), jnp.float32),
    )(x, w, b)


def _tc_stage(body, p, w, b):
    return pl.pallas_call(
        body,
        grid=(GRID,),
        in_specs=[
            pl.BlockSpec((NC, MROWS, HP), lambda i: (0, i, 0)),
            pl.BlockSpec((HP, HP), lambda i: (0, 0)),
            pl.BlockSpec((1, HP), lambda i: (0, 0)),
        ],
        out_specs=pl.BlockSpec((MROWS, HP), lambda i: (i, 0)),
        out_shape=jax.ShapeDtypeStruct((NP, HP), jnp.float32),
    )(p, w, b)


def _spmm_body(h_hbm, src_hbm, dst_hbm, w_hbm, out_hbm,
               h_sh, acc, src_v, dst_v, w_v, rows_a, rows_b, sem_a, sem_b):
    c = lax.axis_index("c")
    s = lax.axis_index("s")
    wid = c * NS + s
    base_rows = s * ROWS_PER_TILE

    pltpu.sync_copy(src_hbm.at[wid], src_v)
    pltpu.sync_copy(dst_hbm.at[wid], dst_v)
    pltpu.sync_copy(w_hbm.at[wid], w_v)

    pltpu.sync_copy(h_hbm.at[pl.ds(base_rows, ROWS_PER_TILE)],
                    h_sh.at[pl.ds(base_rows, ROWS_PER_TILE)])

    for j in range(CHUNK):
        rows_a[j, :] = jnp.zeros((HP,), jnp.float32)
    for r in range(ROWS_PER_TILE // CHUNK):
        pltpu.sync_copy(rows_a, acc.at[pl.ds(base_rows + r * CHUNK, CHUNK)])
    plsc.subcore_barrier()

    def _scale(buf, i):
        for j16 in range(0, CHUNK, HP):
            w16 = w_v[i, pl.ds(j16, HP)]
            for jj in range(HP):
                buf[j16 + jj, :] = buf[j16 + jj, :] * w16[jj]

    pltpu.async_copy(h_sh.at[src_v.at[0]], rows_a, sem_a)

    def _step(k, carry):
        i0 = 2 * k
        pltpu.make_async_copy(h_sh.at[src_v.at[i0]], rows_a, sem_a).wait()
        pltpu.async_copy(h_sh.at[src_v.at[i0 + 1]], rows_b, sem_b)
        _scale(rows_a, i0)
        pltpu.sync_copy(rows_a, acc.at[dst_v.at[i0]], add=True)
        pltpu.make_async_copy(h_sh.at[src_v.at[i0 + 1]], rows_b, sem_b).wait()
        pltpu.async_copy(h_sh.at[src_v.at[i0 + 2]], rows_a, sem_a)
        _scale(rows_b, i0 + 1)
        pltpu.sync_copy(rows_b, acc.at[dst_v.at[i0 + 1]], add=True)
        return carry

    lax.fori_loop(0, NCHUNK // 2, _step, 0)
    pltpu.make_async_copy(h_sh.at[src_v.at[NCHUNK]], rows_a, sem_a).wait()
    plsc.subcore_barrier()

    pltpu.sync_copy(acc.at[pl.ds(base_rows, ROWS_PER_TILE)],
                    out_hbm.at[c, pl.ds(base_rows, ROWS_PER_TILE)])


@functools.partial(
    pl.kernel,
    mesh=plsc.VectorSubcoreMesh(core_axis_name="c", subcore_axis_name="s"),
    compiler_params=pltpu.CompilerParams(use_tc_tiling_on_sc=False),
    out_type=jax.ShapeDtypeStruct((NC, NP, HP), jnp.float32),
    scratch_types=[
        pltpu.VMEM_SHARED((NP, HP), jnp.float32),
        pltpu.VMEM_SHARED((NP, HP), jnp.float32),
        pltpu.VMEM((NCHUNK + 1, CHUNK), jnp.int32),
        pltpu.VMEM((NCHUNK, CHUNK), jnp.int32),
        pltpu.VMEM((NCHUNK, CHUNK), jnp.float32),
        pltpu.VMEM((CHUNK, HP), jnp.float32),
        pltpu.VMEM((CHUNK, HP), jnp.float32),
        pltpu.SemaphoreType.DMA,
        pltpu.SemaphoreType.DMA,
    ],
)
def _spmm(h_hbm, src_hbm, dst_hbm, w_hbm, out_hbm,
          h_sh, acc, src_v, dst_v, w_v, rows_a, rows_b, sem_a, sem_b):
    _spmm_body(h_hbm, src_hbm, dst_hbm, w_hbm, out_hbm,
               h_sh, acc, src_v, dst_v, w_v, rows_a, rows_b, sem_a, sem_b)


def kernel(x, edge_index, edge_weight, W1, b1, W2, b2, Wd, bd):
    f32 = jnp.float32
    W1p = jnp.zeros((D, HP), f32).at[:, :H].set(W1)
    b1p = jnp.zeros((1, HP), f32).at[0, :H].set(b1)
    W2p = jnp.zeros((HP, HP), f32).at[:H, :H].set(W2)
    b2p = jnp.zeros((1, HP), f32).at[0, :H].set(b2)
    Wdp = jnp.zeros((HP, HP), f32).at[:H, : 2 * LATENT].set(Wd)
    bdp = jnp.zeros((1, HP), f32).at[0, : 2 * LATENT].set(bd)

    epw = EDGES_PER_WORKER
    src3 = jnp.pad(edge_index[0].reshape(NW, epw),
                   ((0, 0), (0, (NCHUNK + 1) * CHUNK - epw))
                   ).reshape(NW, NCHUNK + 1, CHUNK)
    dst3 = jnp.pad(edge_index[1].reshape(NW, epw),
                   ((0, 0), (0, EPW_PAD - epw))).reshape(NW, NCHUNK, CHUNK)
    w3 = jnp.pad(edge_weight.reshape(NW, epw),
                 ((0, 0), (0, EPW_PAD - epw))).reshape(NW, NCHUNK, CHUNK)

    h1 = _dense1(x, W1p, b1p)
    p1 = _spmm(h1, src3, dst3, w3)
    h2 = _tc_stage(_combine_body, p1, W2p, b2p)
    p2 = _spmm(h2, src3, dst3, w3)
    out = _tc_stage(_head_body, p2, Wdp, bdp)
    return out[:N, :LATENT], out[:N, LATENT : 2 * LATENT]


# speedup vs baseline: 22.1621x; 1.1439x over previous
import functools

import jax
import jax.numpy as jnp
import numpy as np
from jax import lax
from jax.experimental import pallas as pl
from jax.experimental.pallas import tpu as pltpu
from jax.experimental.pallas import tpu_sc as plsc

N = 10000
E = 320000
D = 128
H = 14
HP = 16
LATENT = 7
SOFTPLUS_INV_1 = float(np.log(np.expm1(1.0)))

NC = 2
NS = 16
NW = NC * NS
EPW = E // NW
CHUNK = 128
NCHUNK = 80
NP = 10240
ROWS_PER_TILE = NP // NS

MROWS = 1000
GRID = N // MROWS


def _dense_body(x_ref, w_ref, b_ref, o_ref):
    o_ref[...] = (
        jnp.dot(x_ref[...], w_ref[...], preferred_element_type=jnp.float32)
        + b_ref[...]
    )


def _combine_body(p_ref, w_ref, b_ref, o_ref):
    a = jnp.maximum(p_ref[0] + p_ref[1], 0.0)
    o_ref[...] = (
        jnp.dot(a, w_ref[...], preferred_element_type=jnp.float32) + b_ref[...]
    )


def _head_body(p_ref, w_ref, b_ref, loc_ref, scale_ref):
    a = jnp.maximum(p_ref[0] + p_ref[1], 0.0)
    lat = jnp.dot(a, w_ref[...], preferred_element_type=jnp.float32) + b_ref[...]
    loc_ref[...] = lat[:, :LATENT]
    z = lat[:, LATENT : 2 * LATENT] + SOFTPLUS_INV_1
    scale_ref[...] = jnp.maximum(z, 0.0) + jnp.log1p(jnp.exp(-jnp.abs(z)))


def _dense1(x, w, b):
    return pl.pallas_call(
        _dense_body,
        grid=(GRID,),
        in_specs=[
            pl.BlockSpec((MROWS, D), lambda i: (i, 0)),
            pl.BlockSpec((D, HP), lambda i: (0, 0)),
            pl.BlockSpec((1, HP), lambda i: (0, 0)),
        ],
        out_specs=pl.BlockSpec((MROWS, HP), lambda i: (i, 0)),
        out_shape=jax.ShapeDtypeStruct((NP, HP), jnp.float32),
    )(x, w, b)


def _combine(p, w, b):
    return pl.pallas_call(
        _combine_body,
        grid=(GRID,),
        in_specs=[
            pl.BlockSpec((NC, MROWS, HP), lambda i: (0, i, 0)),
            pl.BlockSpec((HP, HP), lambda i: (0, 0)),
            pl.BlockSpec((1, HP), lambda i: (0, 0)),
        ],
        out_specs=pl.BlockSpec((MROWS, HP), lambda i: (i, 0)),
        out_shape=jax.ShapeDtypeStruct((NP, HP), jnp.float32),
    )(p, w, b)


def _head(p, w, b):
    return pl.pallas_call(
        _head_body,
        grid=(GRID,),
        in_specs=[
            pl.BlockSpec((NC, MROWS, HP), lambda i: (0, i, 0)),
            pl.BlockSpec((HP, HP), lambda i: (0, 0)),
            pl.BlockSpec((1, HP), lambda i: (0, 0)),
        ],
        out_specs=[
            pl.BlockSpec((MROWS, LATENT), lambda i: (i, 0)),
            pl.BlockSpec((MROWS, LATENT), lambda i: (i, 0)),
        ],
        out_shape=[
            jax.ShapeDtypeStruct((N, LATENT), jnp.float32),
            jax.ShapeDtypeStruct((N, LATENT), jnp.float32),
        ],
    )(p, w, b)


def _spmm_body(h_hbm, ei_hbm, w_hbm, out_hbm,
               h_sh, acc, src_v, dst_v, w_v, rows_a, rows_b, sem_a, sem_b):
    c = lax.axis_index("c")
    s = lax.axis_index("s")
    wid = c * NS + s
    base_rows = s * ROWS_PER_TILE
    ebase = wid * EPW

    zi = jnp.zeros((HP,), jnp.int32)
    zf = jnp.zeros((HP,), jnp.float32)
    for t in range(EPW, (NCHUNK + 1) * CHUNK, HP):
        src_v[pl.ds(t, HP)] = zi
    for t in range(EPW, NCHUNK * CHUNK, HP):
        dst_v[pl.ds(t, HP)] = zi
        w_v[pl.ds(t, HP)] = zf

    pltpu.sync_copy(ei_hbm.at[0, pl.ds(ebase, EPW)], src_v.at[pl.ds(0, EPW)])
    pltpu.sync_copy(ei_hbm.at[1, pl.ds(ebase, EPW)], dst_v.at[pl.ds(0, EPW)])
    pltpu.sync_copy(w_hbm.at[pl.ds(ebase, EPW)], w_v.at[pl.ds(0, EPW)])

    pltpu.sync_copy(h_hbm.at[pl.ds(base_rows, ROWS_PER_TILE)],
                    h_sh.at[pl.ds(base_rows, ROWS_PER_TILE)])

    for j in range(CHUNK):
        rows_a[j, :] = jnp.zeros((HP,), jnp.float32)
    for r in range(ROWS_PER_TILE // CHUNK):
        pltpu.sync_copy(rows_a, acc.at[pl.ds(base_rows + r * CHUNK, CHUNK)])
    plsc.subcore_barrier()

    def _scale(buf, e0):
        for j16 in range(0, CHUNK, HP):
            w16 = w_v[pl.ds(e0 + j16, HP)]
            for jj in range(HP):
                buf[j16 + jj, :] = buf[j16 + jj, :] * w16[jj]

    pltpu.async_copy(h_sh.at[src_v.at[pl.ds(0, CHUNK)]], rows_a, sem_a)

    def _step(k, carry):
        e0 = 2 * k * CHUNK
        e1 = e0 + CHUNK
        e2 = e0 + 2 * CHUNK
        pltpu.make_async_copy(
            h_sh.at[src_v.at[pl.ds(e0, CHUNK)]], rows_a, sem_a).wait()
        pltpu.async_copy(h_sh.at[src_v.at[pl.ds(e1, CHUNK)]], rows_b, sem_b)
        _scale(rows_a, e0)
        pltpu.sync_copy(rows_a, acc.at[dst_v.at[pl.ds(e0, CHUNK)]], add=True)
        pltpu.make_async_copy(
            h_sh.at[src_v.at[pl.ds(e1, CHUNK)]], rows_b, sem_b).wait()
        pltpu.async_copy(h_sh.at[src_v.at[pl.ds(e2, CHUNK)]], rows_a, sem_a)
        _scale(rows_b, e1)
        pltpu.sync_copy(rows_b, acc.at[dst_v.at[pl.ds(e1, CHUNK)]], add=True)
        return carry

    lax.fori_loop(0, NCHUNK // 2, _step, 0)
    pltpu.make_async_copy(
        h_sh.at[src_v.at[pl.ds(NCHUNK * CHUNK, CHUNK)]], rows_a, sem_a).wait()
    plsc.subcore_barrier()

    pltpu.sync_copy(acc.at[pl.ds(base_rows, ROWS_PER_TILE)],
                    out_hbm.at[c, pl.ds(base_rows, ROWS_PER_TILE)])


@functools.partial(
    pl.kernel,
    mesh=plsc.VectorSubcoreMesh(core_axis_name="c", subcore_axis_name="s"),
    compiler_params=pltpu.CompilerParams(use_tc_tiling_on_sc=False),
    out_type=jax.ShapeDtypeStruct((NC, NP, HP), jnp.float32),
    scratch_types=[
        pltpu.VMEM_SHARED((NP, HP), jnp.float32),
        pltpu.VMEM_SHARED((NP, HP), jnp.float32),
        pltpu.VMEM(((NCHUNK + 1) * CHUNK,), jnp.int32),
        pltpu.VMEM((NCHUNK * CHUNK,), jnp.int32),
        pltpu.VMEM((NCHUNK * CHUNK,), jnp.float32),
        pltpu.VMEM((CHUNK, HP), jnp.float32),
        pltpu.VMEM((CHUNK, HP), jnp.float32),
        pltpu.SemaphoreType.DMA,
        pltpu.SemaphoreType.DMA,
    ],
)
def _spmm(h_hbm, ei_hbm, w_hbm, out_hbm,
          h_sh, acc, src_v, dst_v, w_v, rows_a, rows_b, sem_a, sem_b):
    _spmm_body(h_hbm, ei_hbm, w_hbm, out_hbm,
               h_sh, acc, src_v, dst_v, w_v, rows_a, rows_b, sem_a, sem_b)


def kernel(x, edge_index, edge_weight, W1, b1, W2, b2, Wd, bd):
    f32 = jnp.float32
    W1p = jnp.zeros((D, HP), f32).at[:, :H].set(W1)
    b1p = jnp.zeros((1, HP), f32).at[0, :H].set(b1)
    W2p = jnp.zeros((HP, HP), f32).at[:H, :H].set(W2)
    b2p = jnp.zeros((1, HP), f32).at[0, :H].set(b2)
    Wdp = jnp.zeros((HP, HP), f32).at[:H, : 2 * LATENT].set(Wd)
    bdp = jnp.zeros((1, HP), f32).at[0, : 2 * LATENT].set(bd)

    h1 = _dense1(x, W1p, b1p)
    p1 = _spmm(h1, edge_index, edge_weight)
    h2 = _combine(p1, W2p, b2p)
    p2 = _spmm(h2, edge_index, edge_weight)
    loc, scale = _head(p2, Wdp, bdp)
    return loc, scale


# speedup vs baseline: 23.2995x; 1.0513x over previous
import functools

import jax
import jax.numpy as jnp
import numpy as np
from jax import lax
from jax.experimental import pallas as pl
from jax.experimental.pallas import tpu as pltpu
from jax.experimental.pallas import tpu_sc as plsc

N = 10000
E = 320000
D = 128
H = 14
HP = 16
LATENT = 7
SOFTPLUS_INV_1 = float(np.log(np.expm1(1.0)))

NC = 2
NS = 16
NW = NC * NS
EPW = E // NW
CHUNK = 128
NCHUNK = 80
NP = 10240
ROWS_PER_TILE = NP // NS

MROWS = 2000
GRID = N // MROWS


def _dense_body(x_ref, w_ref, b_ref, o_ref):
    o_ref[...] = (
        jnp.dot(x_ref[...], w_ref[...], preferred_element_type=jnp.float32)
        + b_ref[...]
    )


def _combine_body(p_ref, w_ref, b_ref, o_ref):
    a = jnp.maximum(p_ref[0] + p_ref[1], 0.0)
    o_ref[...] = (
        jnp.dot(a, w_ref[...], preferred_element_type=jnp.float32) + b_ref[...]
    )


def _head_body(p_ref, w_ref, b_ref, loc_ref, scale_ref):
    a = jnp.maximum(p_ref[0] + p_ref[1], 0.0)
    lat = jnp.dot(a, w_ref[...], preferred_element_type=jnp.float32) + b_ref[...]
    loc_ref[...] = lat[:, :LATENT]
    z = lat[:, LATENT : 2 * LATENT] + SOFTPLUS_INV_1
    scale_ref[...] = jnp.maximum(z, 0.0) + jnp.log1p(jnp.exp(-jnp.abs(z)))


def _dense1(x, w, b):
    return pl.pallas_call(
        _dense_body,
        grid=(GRID,),
        in_specs=[
            pl.BlockSpec((MROWS, D), lambda i: (i, 0)),
            pl.BlockSpec((D, HP), lambda i: (0, 0)),
            pl.BlockSpec((1, HP), lambda i: (0, 0)),
        ],
        out_specs=pl.BlockSpec((MROWS, HP), lambda i: (i, 0)),
        out_shape=jax.ShapeDtypeStruct((NP, HP), jnp.float32),
    )(x, w, b)


def _combine(p, w, b):
    return pl.pallas_call(
        _combine_body,
        grid=(GRID,),
        in_specs=[
            pl.BlockSpec((NC, MROWS, HP), lambda i: (0, i, 0)),
            pl.BlockSpec((HP, HP), lambda i: (0, 0)),
            pl.BlockSpec((1, HP), lambda i: (0, 0)),
        ],
        out_specs=pl.BlockSpec((MROWS, HP), lambda i: (i, 0)),
        out_shape=jax.ShapeDtypeStruct((NP, HP), jnp.float32),
    )(p, w, b)


def _head(p, w, b):
    return pl.pallas_call(
        _head_body,
        grid=(GRID,),
        in_specs=[
            pl.BlockSpec((NC, MROWS, HP), lambda i: (0, i, 0)),
            pl.BlockSpec((HP, HP), lambda i: (0, 0)),
            pl.BlockSpec((1, HP), lambda i: (0, 0)),
        ],
        out_specs=[
            pl.BlockSpec((MROWS, LATENT), lambda i: (i, 0)),
            pl.BlockSpec((MROWS, LATENT), lambda i: (i, 0)),
        ],
        out_shape=[
            jax.ShapeDtypeStruct((N, LATENT), jnp.float32),
            jax.ShapeDtypeStruct((N, LATENT), jnp.float32),
        ],
    )(p, w, b)


def _spmm_body(h_hbm, ei_hbm, w_hbm, out_hbm,
               h_sh, acc, src_v, dst_v, w_v, rows_a, rows_b, sem_a, sem_b):
    c = lax.axis_index("c")
    s = lax.axis_index("s")
    wid = c * NS + s
    base_rows = s * ROWS_PER_TILE
    ebase = wid * EPW

    zi = jnp.zeros((HP,), jnp.int32)
    zf = jnp.zeros((HP,), jnp.float32)
    for t in range(EPW, (NCHUNK + 1) * CHUNK, HP):
        src_v[pl.ds(t, HP)] = zi
    for t in range(EPW, NCHUNK * CHUNK, HP):
        dst_v[pl.ds(t, HP)] = zi
        w_v[pl.ds(t, HP)] = zf

    pltpu.sync_copy(ei_hbm.at[0, pl.ds(ebase, EPW)], src_v.at[pl.ds(0, EPW)])
    pltpu.sync_copy(ei_hbm.at[1, pl.ds(ebase, EPW)], dst_v.at[pl.ds(0, EPW)])
    pltpu.sync_copy(w_hbm.at[pl.ds(ebase, EPW)], w_v.at[pl.ds(0, EPW)])

    pltpu.sync_copy(h_hbm.at[pl.ds(base_rows, ROWS_PER_TILE)],
                    h_sh.at[pl.ds(base_rows, ROWS_PER_TILE)])

    for j in range(CHUNK):
        rows_a[j, :] = jnp.zeros((HP,), jnp.float32)
    for r in range(ROWS_PER_TILE // CHUNK):
        pltpu.sync_copy(rows_a, acc.at[pl.ds(base_rows + r * CHUNK, CHUNK)])
    plsc.subcore_barrier()

    def _scale(buf, e0):
        for j16 in range(0, CHUNK, HP):
            w16 = w_v[pl.ds(e0 + j16, HP)]
            for jj in range(HP):
                buf[j16 + jj, :] = buf[j16 + jj, :] * w16[jj]

    pltpu.async_copy(h_sh.at[src_v.at[pl.ds(0, CHUNK)]], rows_a, sem_a)

    def _step(k, carry):
        e0 = 2 * k * CHUNK
        e1 = e0 + CHUNK
        e2 = e0 + 2 * CHUNK
        pltpu.make_async_copy(
            h_sh.at[src_v.at[pl.ds(e0, CHUNK)]], rows_a, sem_a).wait()
        pltpu.async_copy(h_sh.at[src_v.at[pl.ds(e1, CHUNK)]], rows_b, sem_b)
        _scale(rows_a, e0)
        pltpu.sync_copy(rows_a, acc.at[dst_v.at[pl.ds(e0, CHUNK)]], add=True)
        pltpu.make_async_copy(
            h_sh.at[src_v.at[pl.ds(e1, CHUNK)]], rows_b, sem_b).wait()
        pltpu.async_copy(h_sh.at[src_v.at[pl.ds(e2, CHUNK)]], rows_a, sem_a)
        _scale(rows_b, e1)
        pltpu.sync_copy(rows_b, acc.at[dst_v.at[pl.ds(e1, CHUNK)]], add=True)
        return carry

    lax.fori_loop(0, NCHUNK // 2, _step, 0)
    pltpu.make_async_copy(
        h_sh.at[src_v.at[pl.ds(NCHUNK * CHUNK, CHUNK)]], rows_a, sem_a).wait()
    plsc.subcore_barrier()

    pltpu.sync_copy(acc.at[pl.ds(base_rows, ROWS_PER_TILE)],
                    out_hbm.at[c, pl.ds(base_rows, ROWS_PER_TILE)])


@functools.partial(
    pl.kernel,
    mesh=plsc.VectorSubcoreMesh(core_axis_name="c", subcore_axis_name="s"),
    compiler_params=pltpu.CompilerParams(use_tc_tiling_on_sc=False),
    out_type=jax.ShapeDtypeStruct((NC, NP, HP), jnp.float32),
    scratch_types=[
        pltpu.VMEM_SHARED((NP, HP), jnp.float32),
        pltpu.VMEM_SHARED((NP, HP), jnp.float32),
        pltpu.VMEM(((NCHUNK + 1) * CHUNK,), jnp.int32),
        pltpu.VMEM((NCHUNK * CHUNK,), jnp.int32),
        pltpu.VMEM((NCHUNK * CHUNK,), jnp.float32),
        pltpu.VMEM((CHUNK, HP), jnp.float32),
        pltpu.VMEM((CHUNK, HP), jnp.float32),
        pltpu.SemaphoreType.DMA,
        pltpu.SemaphoreType.DMA,
    ],
)
def _spmm(h_hbm, ei_hbm, w_hbm, out_hbm,
          h_sh, acc, src_v, dst_v, w_v, rows_a, rows_b, sem_a, sem_b):
    _spmm_body(h_hbm, ei_hbm, w_hbm, out_hbm,
               h_sh, acc, src_v, dst_v, w_v, rows_a, rows_b, sem_a, sem_b)


def kernel(x, edge_index, edge_weight, W1, b1, W2, b2, Wd, bd):
    f32 = jnp.float32
    W1p = jnp.zeros((D, HP), f32).at[:, :H].set(W1)
    b1p = jnp.zeros((1, HP), f32).at[0, :H].set(b1)
    W2p = jnp.zeros((HP, HP), f32).at[:H, :H].set(W2)
    b2p = jnp.zeros((1, HP), f32).at[0, :H].set(b2)
    Wdp = jnp.zeros((HP, HP), f32).at[:H, : 2 * LATENT].set(Wd)
    bdp = jnp.zeros((1, HP), f32).at[0, : 2 * LATENT].set(bd)

    h1 = _dense1(x, W1p, b1p)
    p1 = _spmm(h1, edge_index, edge_weight)
    h2 = _combine(p1, W2p, b2p)
    p2 = _spmm(h2, edge_index, edge_weight)
    loc, scale = _head(p2, Wdp, bdp)
    return loc, scale
